# baseline (speedup 1.0000x reference)
import functools
import jax
import jax.numpy as jnp
from jax import lax
from jax.experimental import pallas as pl
from jax.experimental.pallas import tpu as pltpu
from jax.experimental.pallas import tpu_sc as plsc

_C = 121
_CP = 128
_B = 1024
_F = 512
_BANK = 200
_NS = 16
_SPW = _B // _NS
_NSL = _F // 16


def _onehot16(j):
    io = lax.broadcasted_iota(jnp.int32, (16,), 0)
    return jnp.where(io == j, 1.0, 0.0).astype(jnp.float32)


def kernel(x, labels, centers, cache_mtx, update_mtx):
    mesh = plsc.VectorSubcoreMesh(core_axis_name="c", subcore_axis_name="s")

    @functools.partial(
        pl.kernel, mesh=mesh,
        out_type=(
            jax.ShapeDtypeStruct((_CP,), jnp.float32),
            jax.ShapeDtypeStruct((16,), jnp.float32),
            jax.ShapeDtypeStruct((2 * _CP, _F), jnp.float32),
        ),
        scratch_types=[
            pltpu.VMEM((_SPW + 16,), jnp.int32),
            pltpu.VMEM((_SPW + 16,), jnp.int32),
            pltpu.VMEM((_SPW,), jnp.int32),
            pltpu.VMEM((_SPW,), jnp.int32),
            pltpu.VMEM((_SPW,), jnp.int32),
            pltpu.VMEM((_SPW, _F), jnp.float32),
            pltpu.VMEM((_SPW, _F), jnp.float32),
            pltpu.VMEM((_SPW,), jnp.float32),
            pltpu.VMEM((_SPW,), jnp.float32),
            pltpu.VMEM((_CP,), jnp.float32),
            pltpu.VMEM((_NS * _CP,), jnp.float32),
            pltpu.VMEM((3, _F), jnp.float32),
            pltpu.VMEM((_F,), jnp.float32),
            pltpu.VMEM((16,), jnp.float32),
            pltpu.VMEM((_CP,), jnp.float32),
            pltpu.VMEM((_CP,), jnp.float32),
            pltpu.VMEM((256,), jnp.float32),
            pltpu.VMEM((16,), jnp.float32),
            pltpu.SMEM((_CP,), jnp.int32),
            pltpu.VMEM_SHARED((_CP,), jnp.float32),
            pltpu.VMEM_SHARED((_NS * _CP,), jnp.float32),
            pltpu.VMEM_SHARED((_CP,), jnp.float32),
            pltpu.VMEM_SHARED((256,), jnp.float32),
            pltpu.SemaphoreType.DMA,
            pltpu.SemaphoreType.DMA,
        ],
    )
    def k(lab_hbm, x_hbm, cen_hbm, w_out, loss_out, mc_hbm,
          rawp_v, idxp_v, idxg_v, idx2_v, idxh_v, x_v, rows_v, val_v,
          ones_v, zro_v, cnt_v, cen_v, mcrow_v, mcn16_v, mcnrd_v, w_v,
          ltbl_v, lout_v, run_sm, seg_sh, cntf_sh, mcn_sh, ltbl_sh,
          sem_x, sem_g):
        cid = lax.axis_index("c")
        sid = lax.axis_index("s")
        base = sid * _SPW
        lane_iota = lax.broadcasted_iota(jnp.int32, (16,), 0)
        gd = lax.GatherDimensionNumbers(
            offset_dims=(), collapsed_slice_dims=(0,), start_index_map=(0,))

        def hsum(a):
            for step in (8, 4, 2, 1):
                idx = jnp.bitwise_xor(lane_iota, step)
                shuf = lax.gather(
                    a, idx[:, None], gd, slice_sizes=(1,),
                    mode=lax.GatherScatterMode.PROMISE_IN_BOUNDS)
                a = a + shuf
            return a

        def vsqrt(d2):
            bits = lax.bitcast_convert_type(d2, jnp.int32)
            y = lax.bitcast_convert_type(
                (bits >> 1) + jnp.int32(0x1FBD1DF5), jnp.float32)
            for _ in range(3):
                y = 0.5 * (y + d2 / y)
            return jnp.where(d2 > 0.0, y, 0.0)

        cp_x = pltpu.async_copy(
            x_hbm.at[pl.ds(base, _SPW)], x_v, sem_x)
        pltpu.sync_copy(lab_hbm.at[pl.ds(base, _SPW)],
                        rawp_v.at[pl.ds(0, _SPW)])
        zl = jnp.zeros((16,), jnp.int32)
        for g in range(_SPW // 16):
            raw = rawp_v[pl.ds(g * 16, 16)] - 40
            wr = jnp.where(raw < 0, raw + _C, raw)
            rawp_v[pl.ds(g * 16, 16)] = raw
            idxp_v[pl.ds(g * 16, 16)] = wr
            idxg_v[pl.ds(g * 16, 16)] = wr
            idx2_v[pl.ds(g * 16, 16)] = wr + sid * _CP
            idxh_v[pl.ds(g * 16, 16)] = wr + cid * _CP
            ones_v[pl.ds(g * 16, 16)] = jnp.ones((16,), jnp.float32)
        rawp_v[pl.ds(_SPW, 16)] = zl
        idxp_v[pl.ds(_SPW, 16)] = zl
        zf = jnp.zeros((16,), jnp.float32)
        for g in range(_CP // 16):
            zro_v[pl.ds(g * 16, 16)] = zf
        pltpu.sync_copy(zro_v, cntf_sh.at[pl.ds(sid * _CP, _CP)])

        @pl.when(sid == 0)
        def _():
            pltpu.sync_copy(zro_v, seg_sh)

        @pl.when(sid < 8)
        def _():
            mcn16 = zf
            for j in range(16):
                c = sid * 16 + j
                cc = jnp.minimum(c, _C - 1)
                live = (c < _C).astype(jnp.float32)
                livef = jnp.full((16,), live, jnp.float32)
                pltpu.sync_copy(cen_hbm.at[cc], cen_v)
                acc = zf
                for kk in range(_NSL):
                    sl = pl.ds(kk * 16, 16)
                    m = (cen_v[0, sl] + cen_v[1, sl] + cen_v[2, sl]) * (
                        1.0 / 3.0) * livef
                    mcrow_v[sl] = m
                    acc = acc + m * m
                mcn = vsqrt(hsum(acc)) * livef
                mcn16 = mcn16 + (mcn - mcn16) * _onehot16(j)

                @pl.when(c < _C)
                def _():
                    pltpu.sync_copy(mcrow_v, mc_hbm.at[cid * _CP + cc])
            mcn16_v[...] = mcn16
            pltpu.sync_copy(mcn16_v, mcn_sh.at[pl.ds(sid * 16, 16)])

        plsc.subcore_barrier()
        pltpu.sync_copy(ones_v, cntf_sh.at[idx2_v], add=True)
        plsc.subcore_barrier()

        cp_g = pltpu.async_copy(mc_hbm.at[idxh_v], rows_v, sem_g)
        pltpu.sync_copy(cntf_sh, cnt_v)
        basev = [zf] * (_CP // 16)
        gcnt = [zf] * (_CP // 16)
        for w in range(_NS):
            pref = jnp.full((16,), (w < sid).astype(jnp.float32),
                            jnp.float32)
            for g in range(_CP // 16):
                row = cnt_v[pl.ds(w * _CP + g * 16, 16)]
                basev[g] = basev[g] + row * pref
                gcnt[g] = gcnt[g] + row
        for g in range(_CP // 16):
            for j in range(16):
                run_sm[g * 16 + j] = basev[g][j].astype(jnp.int32)
        cp_x.wait()
        cp_g.wait()

        def body(s, carry):
            v0, v1, v2, v3, lossacc, nvacc = carry
            l = idxp_v[pl.ds(s, 16)][0]
            rw = rawp_v[pl.ds(s, 16)][0]
            r = run_sm[l]
            run_sm[l] = r + 1
            a = zf
            for kk in range(_NSL):
                sl = pl.ds(kk * 16, 16)
                d = x_v[s, sl] - rows_v[s, sl]
                a = a + d * d
            d2 = hsum(a)
            nrm = vsqrt(d2)
            incf = jnp.full((16,), (r < _BANK).astype(jnp.float32),
                            jnp.float32)
            validf = jnp.full((16,), (rw >= 0).astype(jnp.float32),
                              jnp.float32)
            lossacc = lossacc + jnp.clip(d2, 1e-12, 1e12) * validf
            nvacc = nvacc + validf
            val = nrm * incf
            lane = lax.rem(s, 16)
            mif = jnp.where(
                lane_iota == jnp.full((16,), lane, jnp.int32), 1.0, 0.0)
            grp = lax.div(s, 16)
            m0 = mif * jnp.full((16,), (grp == 0).astype(jnp.float32),
                                jnp.float32)
            m1 = mif * jnp.full((16,), (grp == 1).astype(jnp.float32),
                                jnp.float32)
            m2 = mif * jnp.full((16,), (grp == 2).astype(jnp.float32),
                                jnp.float32)
            m3 = mif * jnp.full((16,), (grp == 3).astype(jnp.float32),
                                jnp.float32)
            v0 = v0 + (val - v0) * m0
            v1 = v1 + (val - v1) * m1
            v2 = v2 + (val - v2) * m2
            v3 = v3 + (val - v3) * m3
            return v0, v1, v2, v3, lossacc, nvacc

        v0, v1, v2, v3, lossacc, nvacc = lax.fori_loop(
            0, _SPW, body, (zf, zf, zf, zf, zf, zf))
        val_v[pl.ds(0, 16)] = v0
        val_v[pl.ds(16, 16)] = v1
        val_v[pl.ds(32, 16)] = v2
        val_v[pl.ds(48, 16)] = v3
        pltpu.sync_copy(val_v, seg_sh.at[idxg_v], add=True)
        lout_v[...] = lossacc * _onehot16(0) + nvacc * _onehot16(1)
        pltpu.sync_copy(lout_v, ltbl_sh.at[pl.ds(sid * 16, 16)])
        plsc.subcore_barrier()

        @pl.when(cid + sid == 0)
        def _():
            pltpu.sync_copy(seg_sh, w_v)
            pltpu.sync_copy(mcn_sh, mcnrd_v)
            bank = jnp.full((16,), jnp.float32(_BANK), jnp.float32)
            dist = []
            tot = zf
            for g in range(_CP // 16):
                sl = pl.ds(g * 16, 16)
                d = w_v[sl] + (bank - jnp.minimum(gcnt[g], bank)) * (
                    mcnrd_v[sl])
                dist.append(d)
                tot = tot + d
            tsum = hsum(tot)
            inv = 1.0 / tsum
            for g in range(_CP // 16):
                w_v[pl.ds(g * 16, 16)] = dist[g] * inv
            pltpu.sync_copy(w_v, w_out)

        @pl.when((1 - cid) + sid == 0)
        def _():
            pltpu.sync_copy(ltbl_sh, ltbl_v)
            t = zf
            for w in range(_NS):
                t = t + ltbl_v[pl.ds(w * 16, 16)]
            ls = t[0]
            nv = t[1]
            loss = (ls + (_B * _C - nv) * 1e-12) * (1.0 / _B)
            lout_v[...] = jnp.full((16,), loss, jnp.float32)
            pltpu.sync_copy(lout_v, loss_out)

    w, loss_vec, _mc = k(labels, x, centers)
    return loss_vec[0], w[:_C]


# speedup vs baseline: 1.4021x; 1.4021x over previous
import functools
import jax
import jax.numpy as jnp
from jax import lax
from jax.experimental import pallas as pl
from jax.experimental.pallas import tpu as pltpu
from jax.experimental.pallas import tpu_sc as plsc

_C = 121
_CP = 128
_B = 1024
_F = 512
_BANK = 200
_NS = 16
_SPW = _B // _NS
_NSL = _F // 16
_CPS = _CP // _NS


def _onehot16(j):
    io = lax.broadcasted_iota(jnp.int32, (16,), 0)
    return jnp.where(io == j, 1.0, 0.0).astype(jnp.float32)


def kernel(x, labels, centers, cache_mtx, update_mtx):
    mesh = plsc.VectorSubcoreMesh(core_axis_name="c", subcore_axis_name="s")

    @functools.partial(
        pl.kernel, mesh=mesh,
        out_type=(
            jax.ShapeDtypeStruct((_CP,), jnp.float32),
            jax.ShapeDtypeStruct((16,), jnp.float32),
            jax.ShapeDtypeStruct((2 * _CP, _F), jnp.float32),
        ),
        scratch_types=[
            pltpu.VMEM((_SPW + 16,), jnp.int32),
            pltpu.VMEM((_SPW + 16,), jnp.int32),
            pltpu.VMEM((_SPW,), jnp.int32),
            pltpu.VMEM((_SPW,), jnp.int32),
            pltpu.VMEM((_SPW,), jnp.int32),
            pltpu.VMEM((_SPW, _F), jnp.float32),
            pltpu.VMEM((_SPW, _F), jnp.float32),
            pltpu.VMEM((_SPW,), jnp.float32),
            pltpu.VMEM((_SPW,), jnp.float32),
            pltpu.VMEM((_CP,), jnp.float32),
            pltpu.VMEM((_NS * _CP,), jnp.float32),
            pltpu.VMEM((_CPS, 3, _F), jnp.float32),
            pltpu.VMEM((_CPS, _F), jnp.float32),
            pltpu.VMEM((16,), jnp.float32),
            pltpu.VMEM((16,), jnp.int32),
            pltpu.VMEM((_CP,), jnp.float32),
            pltpu.VMEM((_CP,), jnp.float32),
            pltpu.VMEM((256,), jnp.float32),
            pltpu.VMEM((16,), jnp.float32),
            pltpu.SMEM((_CP,), jnp.int32),
            pltpu.VMEM_SHARED((_CP,), jnp.float32),
            pltpu.VMEM_SHARED((_NS * _CP,), jnp.float32),
            pltpu.VMEM_SHARED((_CP,), jnp.float32),
            pltpu.VMEM_SHARED((256,), jnp.float32),
            pltpu.SemaphoreType.DMA,
            pltpu.SemaphoreType.DMA,
            pltpu.SemaphoreType.DMA,
        ],
    )
    def k(lab_hbm, x_hbm, cen_hbm, w_out, loss_out, mc_hbm,
          rawp_v, idxp_v, idxg_v, idx2_v, idxh_v, x_v, rows_v,
          val_v, ones_v, zro_v, cnt_v, cenblk_v, mcblk_v, mcn16_v,
          midx_v, mcnrd_v, w_v, ltbl_v, lout_v, run_sm, seg_sh, cntf_sh,
          mcn_sh, ltbl_sh, sem_x, sem_g, sem_c):
        cid = lax.axis_index("c")
        sid = lax.axis_index("s")
        base = sid * _SPW
        lane_iota = lax.broadcasted_iota(jnp.int32, (16,), 0)
        gd = lax.GatherDimensionNumbers(
            offset_dims=(), collapsed_slice_dims=(0,), start_index_map=(0,))

        def hsum(a):
            for step in (8, 4, 2, 1):
                idx = jnp.bitwise_xor(lane_iota, step)
                shuf = lax.gather(
                    a, idx[:, None], gd, slice_sizes=(1,),
                    mode=lax.GatherScatterMode.PROMISE_IN_BOUNDS)
                a = a + shuf
            return a

        def vsqrt(d2):
            bits = lax.bitcast_convert_type(d2, jnp.int32)
            y = lax.bitcast_convert_type(
                (bits >> 1) + jnp.int32(0x1FBD1DF5), jnp.float32)
            for _ in range(3):
                y = 0.5 * (y + d2 / y)
            return jnp.where(d2 > 0.0, y, 0.0)

        cp_x = pltpu.async_copy(
            x_hbm.at[pl.ds(base, _SPW)], x_v, sem_x)
        cstart = jnp.minimum(sid * _CPS, _C - _CPS)
        cp_c = pltpu.async_copy(
            cen_hbm.at[pl.ds(cstart, _CPS)], cenblk_v, sem_c)
        pltpu.sync_copy(lab_hbm.at[pl.ds(base, _SPW)],
                        rawp_v.at[pl.ds(0, _SPW)])
        zl = jnp.zeros((16,), jnp.int32)
        for g in range(_SPW // 16):
            raw = rawp_v[pl.ds(g * 16, 16)] - 40
            wr = jnp.where(raw < 0, raw + _C, raw)
            rawp_v[pl.ds(g * 16, 16)] = raw
            idxp_v[pl.ds(g * 16, 16)] = wr
            idxg_v[pl.ds(g * 16, 16)] = wr
            idx2_v[pl.ds(g * 16, 16)] = wr + sid * _CP
            idxh_v[pl.ds(g * 16, 16)] = wr + cid * _CP
            ones_v[pl.ds(g * 16, 16)] = jnp.ones((16,), jnp.float32)
        rawp_v[pl.ds(_SPW, 16)] = zl
        idxp_v[pl.ds(_SPW, 16)] = zl
        zf = jnp.zeros((16,), jnp.float32)
        for g in range(_CP // 16):
            zro_v[pl.ds(g * 16, 16)] = zf
        pltpu.sync_copy(zro_v, cntf_sh.at[pl.ds(sid * _CP, _CP)])

        @pl.when(sid == 0)
        def _():
            pltpu.sync_copy(zro_v, seg_sh)

        cp_c.wait()
        mcn16 = zf
        for j in range(_CPS):
            acc = zf
            for kk in range(_NSL):
                sl = pl.ds(kk * 16, 16)
                m = (cenblk_v[j, 0, sl] + cenblk_v[j, 1, sl]
                     + cenblk_v[j, 2, sl]) * (1.0 / 3.0)
                mcblk_v[j, sl] = m
                acc = acc + m * m
            mcn = vsqrt(hsum(acc))
            mcn16 = mcn16 + (mcn - mcn16) * _onehot16(j)
        mcn16_v[...] = mcn16

        @pl.when(sid < _NS - 1)
        def _():
            pltpu.sync_copy(
                mcblk_v, mc_hbm.at[pl.ds(cid * _CP + sid * _CPS, _CPS)])

        @pl.when(sid == _NS - 1)
        def _():
            pltpu.sync_copy(mcblk_v.at[_CPS - 1],
                            mc_hbm.at[cid * _CP + _C - 1])

        midx_v[...] = jnp.where(lane_iota < _CPS, cstart + lane_iota,
                                jnp.int32(_C))
        pltpu.sync_copy(mcn16_v, mcn_sh.at[midx_v])

        plsc.subcore_barrier()
        pltpu.sync_copy(ones_v, cntf_sh.at[idx2_v], add=True)
        plsc.subcore_barrier()

        cp_g = pltpu.async_copy(mc_hbm.at[idxh_v], rows_v, sem_g)
        pltpu.sync_copy(cntf_sh, cnt_v)
        basev = [zf] * (_CP // 16)
        gcnt = [zf] * (_CP // 16)
        for w in range(_NS):
            pref = jnp.full((16,), (w < sid).astype(jnp.float32),
                            jnp.float32)
            for g in range(_CP // 16):
                row = cnt_v[pl.ds(w * _CP + g * 16, 16)]
                basev[g] = basev[g] + row * pref
                gcnt[g] = gcnt[g] + row
        for g in range(_CP // 16):
            for j in range(16):
                run_sm[g * 16 + j] = basev[g][j].astype(jnp.int32)
        cp_x.wait()
        cp_g.wait()

        def body(s, carry):
            v0, v1, v2, v3, lossacc, nvacc = carry
            l = idxp_v[pl.ds(s, 16)][0]
            rw = rawp_v[pl.ds(s, 16)][0]
            r = run_sm[l]
            run_sm[l] = r + 1
            a = zf
            for kk in range(_NSL):
                sl = pl.ds(kk * 16, 16)
                d = x_v[s, sl] - rows_v[s, sl]
                a = a + d * d
            d2 = hsum(a)
            nrm = vsqrt(d2)
            incf = jnp.full((16,), (r < _BANK).astype(jnp.float32),
                            jnp.float32)
            validf = jnp.full((16,), (rw >= 0).astype(jnp.float32),
                              jnp.float32)
            lossacc = lossacc + jnp.clip(d2, 1e-12, 1e12) * validf
            nvacc = nvacc + validf
            val = nrm * incf
            lane = lax.rem(s, 16)
            mif = jnp.where(
                lane_iota == jnp.full((16,), lane, jnp.int32), 1.0, 0.0)
            grp = lax.div(s, 16)
            m0 = mif * jnp.full((16,), (grp == 0).astype(jnp.float32),
                                jnp.float32)
            m1 = mif * jnp.full((16,), (grp == 1).astype(jnp.float32),
                                jnp.float32)
            m2 = mif * jnp.full((16,), (grp == 2).astype(jnp.float32),
                                jnp.float32)
            m3 = mif * jnp.full((16,), (grp == 3).astype(jnp.float32),
                                jnp.float32)
            v0 = v0 + (val - v0) * m0
            v1 = v1 + (val - v1) * m1
            v2 = v2 + (val - v2) * m2
            v3 = v3 + (val - v3) * m3
            return v0, v1, v2, v3, lossacc, nvacc

        v0, v1, v2, v3, lossacc, nvacc = lax.fori_loop(
            0, _SPW, body, (zf, zf, zf, zf, zf, zf))
        val_v[pl.ds(0, 16)] = v0
        val_v[pl.ds(16, 16)] = v1
        val_v[pl.ds(32, 16)] = v2
        val_v[pl.ds(48, 16)] = v3
        pltpu.sync_copy(val_v, seg_sh.at[idxg_v], add=True)
        lout_v[...] = lossacc * _onehot16(0) + nvacc * _onehot16(1)
        pltpu.sync_copy(lout_v, ltbl_sh.at[pl.ds(sid * 16, 16)])
        plsc.subcore_barrier()

        @pl.when(cid + sid == 0)
        def _():
            pltpu.sync_copy(seg_sh, w_v)
            pltpu.sync_copy(mcn_sh, mcnrd_v)
            bank = jnp.full((16,), jnp.float32(_BANK), jnp.float32)
            dist = []
            tot = zf
            for g in range(_CP // 16):
                sl = pl.ds(g * 16, 16)
                d = w_v[sl] + (bank - jnp.minimum(gcnt[g], bank)) * (
                    mcnrd_v[sl])
                if (g + 1) * 16 > _C:
                    d = jnp.where(lane_iota < _C - g * 16, d, 0.0)
                dist.append(d)
                tot = tot + d
            tsum = hsum(tot)
            inv = 1.0 / tsum
            for g in range(_CP // 16):
                w_v[pl.ds(g * 16, 16)] = dist[g] * inv
            pltpu.sync_copy(w_v, w_out)

        @pl.when((1 - cid) + sid == 0)
        def _():
            pltpu.sync_copy(ltbl_sh, ltbl_v)
            t = zf
            for w in range(_NS):
                t = t + ltbl_v[pl.ds(w * 16, 16)]
            ls = t[0]
            nv = t[1]
            loss = (ls + (_B * _C - nv) * 1e-12) * (1.0 / _B)
            lout_v[...] = jnp.full((16,), loss, jnp.float32)
            pltpu.sync_copy(lout_v, loss_out)

    w, loss_vec, _mc = k(labels, x, centers)
    return loss_vec[0], w[:_C]


# speedup vs baseline: 1.4748x; 1.0518x over previous
import functools
import jax
import jax.numpy as jnp
from jax import lax
from jax.experimental import pallas as pl
from jax.experimental.pallas import tpu as pltpu
from jax.experimental.pallas import tpu_sc as plsc

_C = 121
_CP = 128
_B = 1024
_F = 512
_BANK = 200
_NS = 16
_SPW = _B // _NS
_NSL = _F // 16
_CPS = _CP // _NS


def _onehot16(j):
    io = lax.broadcasted_iota(jnp.int32, (16,), 0)
    return jnp.where(io == j, 1.0, 0.0).astype(jnp.float32)


def kernel(x, labels, centers, cache_mtx, update_mtx):
    mesh = plsc.VectorSubcoreMesh(core_axis_name="c", subcore_axis_name="s")

    @functools.partial(
        pl.kernel, mesh=mesh,
        out_type=(
            jax.ShapeDtypeStruct((_CP,), jnp.float32),
            jax.ShapeDtypeStruct((16,), jnp.float32),
            jax.ShapeDtypeStruct((_CP, _F), jnp.float32),
        ),
        scratch_types=[
            pltpu.VMEM((_SPW + 16,), jnp.int32),
            pltpu.VMEM((_SPW + 16,), jnp.int32),
            pltpu.VMEM((_SPW,), jnp.int32),
            pltpu.VMEM((_SPW,), jnp.int32),
            pltpu.VMEM((_SPW, _F), jnp.float32),
            pltpu.VMEM((_SPW, _F), jnp.float32),
            pltpu.VMEM((_SPW,), jnp.float32),
            pltpu.VMEM((_SPW,), jnp.float32),
            pltpu.VMEM((_CP,), jnp.float32),
            pltpu.VMEM((_NS * _CP,), jnp.float32),
            pltpu.VMEM((_CPS, 3, _F), jnp.float32),
            pltpu.VMEM((_CPS, _F), jnp.float32),
            pltpu.VMEM((16,), jnp.float32),
            pltpu.VMEM((16,), jnp.int32),
            pltpu.VMEM((_CP,), jnp.float32),
            pltpu.VMEM((_CP,), jnp.float32),
            pltpu.VMEM((256,), jnp.float32),
            pltpu.VMEM((16,), jnp.float32),
            pltpu.SMEM((_CP,), jnp.int32),
            pltpu.VMEM_SHARED((_CP,), jnp.float32),
            pltpu.VMEM_SHARED((_NS * _CP,), jnp.float32),
            pltpu.VMEM_SHARED((_CP,), jnp.float32),
            pltpu.VMEM_SHARED((256,), jnp.float32),
            pltpu.SemaphoreType.DMA,
            pltpu.SemaphoreType.DMA,
            pltpu.SemaphoreType.DMA,
        ],
    )
    def k(lab_hbm, x_hbm, cen_hbm, w_out, loss_out, mc_hbm,
          rawp_v, idxp_v, idxg_v, idx2_v, x_v, rows_v,
          val_v, ones_v, zro_v, cnt_v, cenblk_v, mcblk_v, mcn16_v,
          midx_v, mcnrd_v, w_v, ltbl_v, lout_v, run_sm, seg_sh, cntf_sh,
          mcn_sh, ltbl_sh, sem_x, sem_g, sem_c):
        cid = lax.axis_index("c")
        sid = lax.axis_index("s")
        base = sid * _SPW
        lane_iota = lax.broadcasted_iota(jnp.int32, (16,), 0)
        gd = lax.GatherDimensionNumbers(
            offset_dims=(), collapsed_slice_dims=(0,), start_index_map=(0,))

        def hsum(a):
            for step in (8, 4, 2, 1):
                idx = jnp.bitwise_xor(lane_iota, step)
                shuf = lax.gather(
                    a, idx[:, None], gd, slice_sizes=(1,),
                    mode=lax.GatherScatterMode.PROMISE_IN_BOUNDS)
                a = a + shuf
            return a

        def vsqrt(d2):
            bits = lax.bitcast_convert_type(d2, jnp.int32)
            y = lax.bitcast_convert_type(
                (bits >> 1) + jnp.int32(0x1FBD1DF5), jnp.float32)
            for _ in range(3):
                y = 0.5 * (y + d2 / y)
            return jnp.where(d2 > 0.0, y, 0.0)

        @pl.when(cid == 0)
        def _core0():
            cp_x = pltpu.async_copy(
                x_hbm.at[pl.ds(base, _SPW)], x_v, sem_x)
            cstart = jnp.minimum(sid * _CPS, _C - _CPS)
            cp_c = pltpu.async_copy(
                cen_hbm.at[pl.ds(cstart, _CPS)], cenblk_v, sem_c)
            pltpu.sync_copy(lab_hbm.at[pl.ds(base, _SPW)],
                            rawp_v.at[pl.ds(0, _SPW)])
            zl = jnp.zeros((16,), jnp.int32)
            for g in range(_SPW // 16):
                raw = rawp_v[pl.ds(g * 16, 16)] - 40
                wr = jnp.where(raw < 0, raw + _C, raw)
                rawp_v[pl.ds(g * 16, 16)] = raw
                idxp_v[pl.ds(g * 16, 16)] = wr
                idxg_v[pl.ds(g * 16, 16)] = wr
                idx2_v[pl.ds(g * 16, 16)] = wr + sid * _CP
                ones_v[pl.ds(g * 16, 16)] = jnp.ones((16,), jnp.float32)
            rawp_v[pl.ds(_SPW, 16)] = zl
            idxp_v[pl.ds(_SPW, 16)] = zl
            zf = jnp.zeros((16,), jnp.float32)
            for g in range(_CP // 16):
                zro_v[pl.ds(g * 16, 16)] = zf
            pltpu.sync_copy(zro_v, cntf_sh.at[pl.ds(sid * _CP, _CP)])

            @pl.when(sid == 0)
            def _():
                pltpu.sync_copy(zro_v, seg_sh)

            cp_c.wait()
            mcn16 = zf
            for j in range(_CPS):
                acc = zf
                for kk in range(_NSL):
                    sl = pl.ds(kk * 16, 16)
                    m = (cenblk_v[j, 0, sl] + cenblk_v[j, 1, sl]
                         + cenblk_v[j, 2, sl]) * (1.0 / 3.0)
                    mcblk_v[j, sl] = m
                    acc = acc + m * m
                mcn = vsqrt(hsum(acc))
                mcn16 = mcn16 + (mcn - mcn16) * _onehot16(j)
            mcn16_v[...] = mcn16

            @pl.when(sid < _NS - 1)
            def _():
                pltpu.sync_copy(
                    mcblk_v, mc_hbm.at[pl.ds(sid * _CPS, _CPS)])

            @pl.when(sid == _NS - 1)
            def _():
                pltpu.sync_copy(mcblk_v.at[_CPS - 1],
                                mc_hbm.at[_C - 1])

            midx_v[...] = jnp.where(lane_iota < _CPS, cstart + lane_iota,
                                    jnp.int32(_C))
            pltpu.sync_copy(mcn16_v, mcn_sh.at[midx_v])

            plsc.subcore_barrier()
            pltpu.sync_copy(ones_v, cntf_sh.at[idx2_v], add=True)
            plsc.subcore_barrier()

            cp_g = pltpu.async_copy(mc_hbm.at[idxg_v], rows_v, sem_g)
            pltpu.sync_copy(cntf_sh, cnt_v)
            basev = [zf] * (_CP // 16)
            gcnt = [zf] * (_CP // 16)
            for w in range(_NS):
                pref = jnp.full((16,), (w < sid).astype(jnp.float32),
                                jnp.float32)
                for g in range(_CP // 16):
                    row = cnt_v[pl.ds(w * _CP + g * 16, 16)]
                    basev[g] = basev[g] + row * pref
                    gcnt[g] = gcnt[g] + row
            for g in range(_CP // 16):
                for j in range(16):
                    run_sm[g * 16 + j] = basev[g][j].astype(jnp.int32)
            cp_x.wait()
            cp_g.wait()

            def body(s, carry):
                v0, v1, v2, v3, lossacc, nvacc = carry
                l = idxp_v[pl.ds(s, 16)][0]
                rw = rawp_v[pl.ds(s, 16)][0]
                r = run_sm[l]
                run_sm[l] = r + 1
                a = zf
                for kk in range(_NSL):
                    sl = pl.ds(kk * 16, 16)
                    d = x_v[s, sl] - rows_v[s, sl]
                    a = a + d * d
                d2 = hsum(a)
                nrm = vsqrt(d2)
                incf = jnp.full((16,), (r < _BANK).astype(jnp.float32),
                                jnp.float32)
                validf = jnp.full((16,), (rw >= 0).astype(jnp.float32),
                                  jnp.float32)
                lossacc = lossacc + jnp.clip(d2, 1e-12, 1e12) * validf
                nvacc = nvacc + validf
                val = nrm * incf
                lane = lax.rem(s, 16)
                mif = jnp.where(
                    lane_iota == jnp.full((16,), lane, jnp.int32),
                    1.0, 0.0)
                grp = lax.div(s, 16)
                m0 = mif * jnp.full((16,), (grp == 0).astype(jnp.float32),
                                    jnp.float32)
                m1 = mif * jnp.full((16,), (grp == 1).astype(jnp.float32),
                                    jnp.float32)
                m2 = mif * jnp.full((16,), (grp == 2).astype(jnp.float32),
                                    jnp.float32)
                m3 = mif * jnp.full((16,), (grp == 3).astype(jnp.float32),
                                    jnp.float32)
                v0 = v0 + (val - v0) * m0
                v1 = v1 + (val - v1) * m1
                v2 = v2 + (val - v2) * m2
                v3 = v3 + (val - v3) * m3
                return v0, v1, v2, v3, lossacc, nvacc

            v0, v1, v2, v3, lossacc, nvacc = lax.fori_loop(
                0, _SPW, body, (zf, zf, zf, zf, zf, zf))
            val_v[pl.ds(0, 16)] = v0
            val_v[pl.ds(16, 16)] = v1
            val_v[pl.ds(32, 16)] = v2
            val_v[pl.ds(48, 16)] = v3
            pltpu.sync_copy(val_v, seg_sh.at[idxg_v], add=True)
            lout_v[...] = lossacc * _onehot16(0) + nvacc * _onehot16(1)
            pltpu.sync_copy(lout_v, ltbl_sh.at[pl.ds(sid * 16, 16)])
            plsc.subcore_barrier()

            @pl.when(sid == 0)
            def _():
                pltpu.sync_copy(seg_sh, w_v)
                pltpu.sync_copy(mcn_sh, mcnrd_v)
                bank = jnp.full((16,), jnp.float32(_BANK), jnp.float32)
                dist = []
                tot = zf
                for g in range(_CP // 16):
                    sl = pl.ds(g * 16, 16)
                    d = w_v[sl] + (bank - jnp.minimum(gcnt[g], bank)) * (
                        mcnrd_v[sl])
                    if (g + 1) * 16 > _C:
                        d = jnp.where(lane_iota < _C - g * 16, d, 0.0)
                    dist.append(d)
                    tot = tot + d
                tsum = hsum(tot)
                inv = 1.0 / tsum
                for g in range(_CP // 16):
                    w_v[pl.ds(g * 16, 16)] = dist[g] * inv
                pltpu.sync_copy(w_v, w_out)

            @pl.when(sid == 1)
            def _():
                pltpu.sync_copy(ltbl_sh, ltbl_v)
                t = zf
                for w in range(_NS):
                    t = t + ltbl_v[pl.ds(w * 16, 16)]
                ls = t[0]
                nv = t[1]
                loss = (ls + (_B * _C - nv) * 1e-12) * (1.0 / _B)
                lout_v[...] = jnp.full((16,), loss, jnp.float32)
                pltpu.sync_copy(lout_v, loss_out)

    w, loss_vec, _mc = k(labels, x, centers)
    return loss_vec[0], w[:_C]


# speedup vs baseline: 1.4839x; 1.0062x over previous
import functools
import jax
import jax.numpy as jnp
from jax import lax
from jax.experimental import pallas as pl
from jax.experimental.pallas import tpu as pltpu
from jax.experimental.pallas import tpu_sc as plsc

_C = 121
_CP = 128
_B = 1024
_F = 512
_BANK = 200
_NS = 16
_SPW = _B // _NS
_NSL = _F // 16
_CPS = _CP // _NS


def _onehot16(j):
    io = lax.broadcasted_iota(jnp.int32, (16,), 0)
    return jnp.where(io == j, 1.0, 0.0).astype(jnp.float32)


def kernel(x, labels, centers, cache_mtx, update_mtx):
    mesh = plsc.VectorSubcoreMesh(core_axis_name="c", subcore_axis_name="s")

    @functools.partial(
        pl.kernel, mesh=mesh,
        out_type=(
            jax.ShapeDtypeStruct((_CP,), jnp.float32),
            jax.ShapeDtypeStruct((16,), jnp.float32),
            jax.ShapeDtypeStruct((_CP, _F), jnp.float32),
        ),
        scratch_types=[
            pltpu.VMEM((_SPW + 16,), jnp.int32),
            pltpu.VMEM((_SPW + 16,), jnp.int32),
            pltpu.VMEM((_SPW,), jnp.int32),
            pltpu.VMEM((_SPW,), jnp.int32),
            pltpu.VMEM((_SPW, _F), jnp.float32),
            pltpu.VMEM((_SPW, _F), jnp.float32),
            pltpu.VMEM((_SPW,), jnp.float32),
            pltpu.VMEM((_SPW,), jnp.float32),
            pltpu.VMEM((_CP,), jnp.float32),
            pltpu.VMEM((_NS * _CP,), jnp.float32),
            pltpu.VMEM((_CPS, 3, _F), jnp.float32),
            pltpu.VMEM((_CPS, _F), jnp.float32),
            pltpu.VMEM((16,), jnp.float32),
            pltpu.VMEM((16,), jnp.int32),
            pltpu.VMEM((_CP,), jnp.float32),
            pltpu.VMEM((_CP,), jnp.float32),
            pltpu.VMEM((256,), jnp.float32),
            pltpu.VMEM((16,), jnp.float32),
            pltpu.SMEM((_CP,), jnp.int32),
            pltpu.VMEM_SHARED((_CP,), jnp.float32),
            pltpu.VMEM_SHARED((_NS * _CP,), jnp.float32),
            pltpu.VMEM_SHARED((_CP,), jnp.float32),
            pltpu.VMEM_SHARED((256,), jnp.float32),
            pltpu.SemaphoreType.DMA,
            pltpu.SemaphoreType.DMA,
            pltpu.SemaphoreType.DMA,
        ],
    )
    def k(lab_hbm, x_hbm, cen_hbm, w_out, loss_out, mc_hbm,
          rawp_v, idxp_v, idxg_v, idx2_v, x_v, rows_v,
          val_v, ones_v, zro_v, cnt_v, cenblk_v, mcblk_v, mcn16_v,
          midx_v, mcnrd_v, w_v, ltbl_v, lout_v, run_sm, seg_sh, cntf_sh,
          mcn_sh, ltbl_sh, sem_x, sem_g, sem_c):
        cid = lax.axis_index("c")
        sid = lax.axis_index("s")
        base = sid * _SPW
        lane_iota = lax.broadcasted_iota(jnp.int32, (16,), 0)
        gd = lax.GatherDimensionNumbers(
            offset_dims=(), collapsed_slice_dims=(0,), start_index_map=(0,))

        def hsum(a):
            for step in (8, 4, 2, 1):
                idx = jnp.bitwise_xor(lane_iota, step)
                shuf = lax.gather(
                    a, idx[:, None], gd, slice_sizes=(1,),
                    mode=lax.GatherScatterMode.PROMISE_IN_BOUNDS)
                a = a + shuf
            return a

        def vsqrt(d2):
            bits = lax.bitcast_convert_type(d2, jnp.int32)
            y = lax.bitcast_convert_type(
                (bits >> 1) + jnp.int32(0x1FBD1DF5), jnp.float32)
            for _ in range(3):
                y = 0.5 * (y + d2 / y)
            return jnp.where(d2 > 0.0, y, 0.0)

        @pl.when(cid == 0)
        def _core0():
            cp_x = pltpu.async_copy(
                x_hbm.at[pl.ds(base, _SPW)], x_v, sem_x)
            cstart = jnp.minimum(sid * _CPS, _C - _CPS)
            cp_c = pltpu.async_copy(
                cen_hbm.at[pl.ds(cstart, _CPS)], cenblk_v, sem_c)
            pltpu.sync_copy(lab_hbm.at[pl.ds(base, _SPW)],
                            rawp_v.at[pl.ds(0, _SPW)])
            zl = jnp.zeros((16,), jnp.int32)
            for g in range(_SPW // 16):
                raw = rawp_v[pl.ds(g * 16, 16)] - 40
                wr = jnp.where(raw < 0, raw + _C, raw)
                rawp_v[pl.ds(g * 16, 16)] = raw
                idxp_v[pl.ds(g * 16, 16)] = wr
                idxg_v[pl.ds(g * 16, 16)] = wr
                idx2_v[pl.ds(g * 16, 16)] = wr + sid * _CP
                ones_v[pl.ds(g * 16, 16)] = jnp.ones((16,), jnp.float32)
            rawp_v[pl.ds(_SPW, 16)] = zl
            idxp_v[pl.ds(_SPW, 16)] = zl
            zf = jnp.zeros((16,), jnp.float32)
            for g in range(_CP // 16):
                zro_v[pl.ds(g * 16, 16)] = zf
            pltpu.sync_copy(zro_v, cntf_sh.at[pl.ds(sid * _CP, _CP)])

            @pl.when(sid == 0)
            def _():
                pltpu.sync_copy(zro_v, seg_sh)

            cp_c.wait()

            def bbody(j, mcn16):
                acc = zf
                for kk in range(_NSL):
                    sl = pl.ds(kk * 16, 16)
                    m = (cenblk_v[j, 0, sl] + cenblk_v[j, 1, sl]
                         + cenblk_v[j, 2, sl]) * (1.0 / 3.0)
                    mcblk_v[j, sl] = m
                    acc = acc + m * m
                mcn = vsqrt(hsum(acc))
                sel = jnp.where(lane_iota == j, 1.0, 0.0).astype(
                    jnp.float32)
                return mcn16 + (mcn - mcn16) * sel

            mcn16_v[...] = lax.fori_loop(0, _CPS, bbody, zf)

            @pl.when(sid < _NS - 1)
            def _():
                pltpu.sync_copy(
                    mcblk_v, mc_hbm.at[pl.ds(sid * _CPS, _CPS)])

            @pl.when(sid == _NS - 1)
            def _():
                pltpu.sync_copy(mcblk_v.at[_CPS - 1],
                                mc_hbm.at[_C - 1])

            midx_v[...] = jnp.where(lane_iota < _CPS, cstart + lane_iota,
                                    jnp.int32(_C))
            pltpu.sync_copy(mcn16_v, mcn_sh.at[midx_v])

            plsc.subcore_barrier()
            pltpu.sync_copy(ones_v, cntf_sh.at[idx2_v], add=True)
            plsc.subcore_barrier()

            cp_g = pltpu.async_copy(mc_hbm.at[idxg_v], rows_v, sem_g)
            pltpu.sync_copy(cntf_sh, cnt_v)
            basev = [zf] * (_CP // 16)
            gcnt = [zf] * (_CP // 16)
            for w in range(_NS):
                pref = jnp.full((16,), (w < sid).astype(jnp.float32),
                                jnp.float32)
                for g in range(_CP // 16):
                    row = cnt_v[pl.ds(w * _CP + g * 16, 16)]
                    basev[g] = basev[g] + row * pref
                    gcnt[g] = gcnt[g] + row
            for g in range(_CP // 16):
                for j in range(16):
                    run_sm[g * 16 + j] = basev[g][j].astype(jnp.int32)
            cp_x.wait()
            cp_g.wait()

            def body(s, carry):
                v0, v1, v2, v3, lossacc, nvacc = carry
                l = idxp_v[pl.ds(s, 16)][0]
                rw = rawp_v[pl.ds(s, 16)][0]
                r = run_sm[l]
                run_sm[l] = r + 1
                a = zf
                for kk in range(_NSL):
                    sl = pl.ds(kk * 16, 16)
                    d = x_v[s, sl] - rows_v[s, sl]
                    a = a + d * d
                d2 = hsum(a)
                nrm = vsqrt(d2)
                incf = jnp.full((16,), (r < _BANK).astype(jnp.float32),
                                jnp.float32)
                validf = jnp.full((16,), (rw >= 0).astype(jnp.float32),
                                  jnp.float32)
                lossacc = lossacc + jnp.clip(d2, 1e-12, 1e12) * validf
                nvacc = nvacc + validf
                val = nrm * incf
                lane = lax.rem(s, 16)
                mif = jnp.where(
                    lane_iota == jnp.full((16,), lane, jnp.int32),
                    1.0, 0.0)
                grp = lax.div(s, 16)
                m0 = mif * jnp.full((16,), (grp == 0).astype(jnp.float32),
                                    jnp.float32)
                m1 = mif * jnp.full((16,), (grp == 1).astype(jnp.float32),
                                    jnp.float32)
                m2 = mif * jnp.full((16,), (grp == 2).astype(jnp.float32),
                                    jnp.float32)
                m3 = mif * jnp.full((16,), (grp == 3).astype(jnp.float32),
                                    jnp.float32)
                v0 = v0 + (val - v0) * m0
                v1 = v1 + (val - v1) * m1
                v2 = v2 + (val - v2) * m2
                v3 = v3 + (val - v3) * m3
                return v0, v1, v2, v3, lossacc, nvacc

            v0, v1, v2, v3, lossacc, nvacc = lax.fori_loop(
                0, _SPW, body, (zf, zf, zf, zf, zf, zf))
            val_v[pl.ds(0, 16)] = v0
            val_v[pl.ds(16, 16)] = v1
            val_v[pl.ds(32, 16)] = v2
            val_v[pl.ds(48, 16)] = v3
            pltpu.sync_copy(val_v, seg_sh.at[idxg_v], add=True)
            lout_v[...] = lossacc * _onehot16(0) + nvacc * _onehot16(1)
            pltpu.sync_copy(lout_v, ltbl_sh.at[pl.ds(sid * 16, 16)])
            plsc.subcore_barrier()

            @pl.when(sid == 0)
            def _():
                pltpu.sync_copy(seg_sh, w_v)
                pltpu.sync_copy(mcn_sh, mcnrd_v)
                bank = jnp.full((16,), jnp.float32(_BANK), jnp.float32)
                dist = []
                tot = zf
                for g in range(_CP // 16):
                    sl = pl.ds(g * 16, 16)
                    d = w_v[sl] + (bank - jnp.minimum(gcnt[g], bank)) * (
                        mcnrd_v[sl])
                    if (g + 1) * 16 > _C:
                        d = jnp.where(lane_iota < _C - g * 16, d, 0.0)
                    dist.append(d)
                    tot = tot + d
                tsum = hsum(tot)
                inv = 1.0 / tsum
                for g in range(_CP // 16):
                    w_v[pl.ds(g * 16, 16)] = dist[g] * inv
                pltpu.sync_copy(w_v, w_out)

            @pl.when(sid == 1)
            def _():
                pltpu.sync_copy(ltbl_sh, ltbl_v)
                t = zf
                for w in range(_NS):
                    t = t + ltbl_v[pl.ds(w * 16, 16)]
                ls = t[0]
                nv = t[1]
                loss = (ls + (_B * _C - nv) * 1e-12) * (1.0 / _B)
                lout_v[...] = jnp.full((16,), loss, jnp.float32)
                pltpu.sync_copy(lout_v, loss_out)

    w, loss_vec, _mc = k(labels, x, centers)
    return loss_vec[0], w[:_C]
